# baseline (speedup 1.0000x reference)
import jax
import jax.numpy as jnp
from jax import lax
from jax.experimental import pallas as pl
from jax.experimental.pallas import tpu as pltpu
from jax.experimental.pallas import tpu_sc as plsc

E = 8
TOP_K = 2
H = 1024
F = 1024
T = 2048
A = TOP_K * T
RBLK = 256
NBLK = A // RBLK + E
NROWS = NBLK * RBLK
LANES = 128
NC = 2
NS = 16
NW = NC * NS
APW = A // NS

_sc_mesh = plsc.VectorSubcoreMesh(
    core_axis_name="c", subcore_axis_name="s", num_cores=NC, num_subcores=NS
)


def _lane16():
    return lax.broadcasted_iota(jnp.int32, (16,), 0)


def _router_body(x_ref, rw_ref, eids_ref, wts_ref):
    rw = rw_ref[...]
    x = x_ref[...]
    logits = lax.dot_general(
        rw, x, (((1,), (1,)), ((), ())), preferred_element_type=jnp.float32
    )
    row = lax.broadcasted_iota(jnp.int32, logits.shape, 0)
    neg = jnp.float32(-1e30)
    logits = jnp.where(row < E, logits, neg)
    m = jnp.max(logits, axis=0, keepdims=True)
    ex = jnp.exp(logits - m)
    ex = jnp.where(row < E, ex, 0.0)
    scores = ex / jnp.sum(ex, axis=0, keepdims=True)
    big = jnp.int32(LANES)
    m1 = jnp.max(scores, axis=0, keepdims=True)
    i1 = jnp.min(jnp.where(scores == m1, row, big), axis=0, keepdims=True)
    sc2 = jnp.where(row == i1, neg, scores)
    m2 = jnp.max(sc2, axis=0, keepdims=True)
    i2 = jnp.min(jnp.where(sc2 == m2, row, big), axis=0, keepdims=True)
    denom = m1 + m2
    krow = lax.broadcasted_iota(jnp.int32, (E, T), 0)
    eids_ref[...] = jnp.where(
        krow == 0, jnp.broadcast_to(i1, (E, T)),
        jnp.where(krow == 1, jnp.broadcast_to(i2, (E, T)), 0),
    )
    wts_ref[...] = jnp.where(
        krow == 0, jnp.broadcast_to(m1 / denom, (E, T)),
        jnp.where(krow == 1, jnp.broadcast_to(m2 / denom, (E, T)), 0.0),
    )


def _router(xf, rw_pad):
    return pl.pallas_call(
        _router_body,
        grid=(1,),
        in_specs=[
            pl.BlockSpec((T, H), lambda i: (0, 0)),
            pl.BlockSpec((LANES, H), lambda i: (0, 0)),
        ],
        out_specs=[
            pl.BlockSpec((E, T), lambda i: (0, 0)),
            pl.BlockSpec((E, T), lambda i: (0, 0)),
        ],
        out_shape=[
            jax.ShapeDtypeStruct((E, T), jnp.int32),
            jax.ShapeDtypeStruct((E, T), jnp.float32),
        ],
    )(xf, rw_pad)


def _meta_body(eids_hbm, sortpos_hbm, stok_hbm, bexp_hbm, nrows_hbm,
               ev, pA, pB, tA, tB, histb, hall, bexb, nrb, zi, shist, stok):
    cid = lax.axis_index("c")
    sid = lax.axis_index("s")
    lane = _lane16()
    zeros16 = jnp.zeros((16,), jnp.int32)

    @pl.when(cid == 0)
    def _():
        for j in range(NROWS // NS // 16):
            zi[pl.ds(j * 16, 16)] = zeros16
        pltpu.sync_copy(zi, stok.at[pl.ds(sid * (NROWS // NS), NROWS // NS)])

    plsc.subcore_barrier()

    @pl.when(cid == 0)
    def _():
        pltpu.sync_copy(eids_hbm.at[pl.ds(sid * APW, APW)], ev)
        cnts = []
        for e in range(E):
            acc = jnp.int32(0)
            for j in range(APW // 16):
                evj = ev[pl.ds(j * 16, 16)]
                acc = acc + jnp.sum(jnp.where(evj == e, 1, 0))
            cnts.append(acc)
        hv = zeros16
        for e in range(E):
            hv = hv + jnp.where(lane == e, cnts[e], 0)
        histb[...] = hv
        pltpu.sync_copy(histb, shist.at[sid])

    plsc.subcore_barrier()

    @pl.when(cid == 0)
    def _():
        pltpu.sync_copy(shist, hall)
        total = zeros16
        base = zeros16
        for w in range(NS):
            hw = hall[w]
            total = total + hw
            base = base + jnp.where(jnp.int32(w) < sid, hw, 0)
        padded = ((total + (RBLK - 1)) >> 8) << 8
        ex_off = plsc.cumsum(padded) - padded
        start = ex_off + base

        cnt = []
        for e in range(E):
            cnt.append(jnp.sum(jnp.where(lane == e, start, 0)))

        @pl.when(sid == 0)
        def _():
            nr = jnp.sum(padded)
            nrb[...] = jnp.full((16,), nr, jnp.int32)
            pltpu.sync_copy(nrb, nrows_hbm)
            off_s = [jnp.sum(jnp.where(lane == e, ex_off, 0)) for e in range(E)]
            pad_s = [jnp.sum(jnp.where(lane == e, padded, 0)) for e in range(E)]
            last_e = jnp.max(jnp.where(padded > 0, lane, 0))
            for v in range(2):
                b = lane + v * 16
                r0 = b * RBLK
                bx = zeros16
                for e in range(E):
                    inside = (r0 >= off_s[e]) & (r0 < off_s[e] + pad_s[e])
                    bx = bx + jnp.where(inside, e, 0)
                bx = jnp.where(r0 < nr, bx, last_e)
                bexb[pl.ds(v * 16, 16)] = bx
            pltpu.sync_copy(bexb, bexp_hbm)

        for j in range(APW // 16):
            evj = ev[pl.ds(j * 16, 16)]
            pos = zeros16
            for e in range(E):
                m = evj == e
                mi = jnp.where(m, 1, 0)
                pref = plsc.cumsum(mi) - mi
                pos = jnp.where(m, cnt[e] + pref, pos)
                cnt[e] = cnt[e] + jnp.sum(mi)
            pbuf, tbuf = (pA, tA) if j < 8 else (pB, tB)
            off = (j % 8) * 16
            pbuf[pl.ds(off, 16)] = pos
            gi = sid * APW + j * 16 + lane
            tbuf[pl.ds(off, 16)] = gi & (T - 1)
        pltpu.sync_copy(pA, sortpos_hbm.at[pl.ds(sid * APW, 128)])
        pltpu.sync_copy(pB, sortpos_hbm.at[pl.ds(sid * APW + 128, 128)])

        pltpu.sync_copy(tA, stok.at[pA])
        pltpu.sync_copy(tB, stok.at[pB])

    plsc.subcore_barrier()

    @pl.when(cid == 0)
    def _():
        pltpu.sync_copy(stok.at[pl.ds(sid * (NROWS // NS), NROWS // NS)], zi)
        pltpu.sync_copy(zi, stok_hbm.at[pl.ds(sid * (NROWS // NS), NROWS // NS)])


_meta = pl.kernel(
    _meta_body,
    out_type=[
        jax.ShapeDtypeStruct((A,), jnp.int32),
        jax.ShapeDtypeStruct((NROWS,), jnp.int32),
        jax.ShapeDtypeStruct((32,), jnp.int32),
        jax.ShapeDtypeStruct((16,), jnp.int32),
    ],
    mesh=_sc_mesh,
    compiler_params=pltpu.CompilerParams(needs_layout_passes=False),
    scratch_types=[
        pltpu.VMEM((APW,), jnp.int32),
        pltpu.VMEM((128,), jnp.int32),
        pltpu.VMEM((128,), jnp.int32),
        pltpu.VMEM((128,), jnp.int32),
        pltpu.VMEM((128,), jnp.int32),
        pltpu.VMEM((16,), jnp.int32),
        pltpu.VMEM((NS, 16), jnp.int32),
        pltpu.VMEM((32,), jnp.int32),
        pltpu.VMEM((16,), jnp.int32),
        pltpu.VMEM((NROWS // NS,), jnp.int32),
        pltpu.VMEM_SHARED((NS, 16), jnp.int32),
        pltpu.VMEM_SHARED((NROWS,), jnp.int32),
    ],
)


GCH = 64


def _gather_body(x_hbm, stok_hbm, xs_hbm, idxb, buf, sem):
    cid = lax.axis_index("c")
    sid = lax.axis_index("s")
    wid = sid * NC + cid
    for j in range(NROWS // GCH // NW):
        base = (wid + j * NW) * GCH
        pltpu.sync_copy(stok_hbm.at[pl.ds(base, GCH)], idxb)
        pltpu.async_copy(x_hbm.at[idxb], buf, sem).wait()
        pltpu.sync_copy(buf, xs_hbm.at[pl.ds(base, GCH)])


_gather = pl.kernel(
    _gather_body,
    out_type=jax.ShapeDtypeStruct((NROWS, H), jnp.float32),
    mesh=_sc_mesh,
    compiler_params=pltpu.CompilerParams(needs_layout_passes=False),
    scratch_types=[
        pltpu.VMEM((GCH,), jnp.int32),
        pltpu.VMEM((GCH, H), jnp.float32),
        pltpu.SemaphoreType.DMA,
    ],
)


def _gemm_body(bexp_ref, xs_ref, w1_ref, v1_ref, w2_ref, y_ref):
    xb = xs_ref[...].astype(jnp.bfloat16)
    w1b = w1_ref[0].astype(jnp.bfloat16)
    v1b = v1_ref[0].astype(jnp.bfloat16)
    w2b = w2_ref[0].astype(jnp.bfloat16)
    h1 = lax.dot_general(
        xb, w1b, (((1,), (1,)), ((), ())), preferred_element_type=jnp.float32
    )
    h2 = lax.dot_general(
        xb, v1b, (((1,), (1,)), ((), ())), preferred_element_type=jnp.float32
    )
    h = (h1 * jax.nn.sigmoid(h1) * h2).astype(jnp.bfloat16)
    y_ref[...] = lax.dot_general(
        h, w2b, (((1,), (0,)), ((), ())), preferred_element_type=jnp.float32
    )


def _gemm(bexp, xs, w1, v1, w2):
    grid_spec = pltpu.PrefetchScalarGridSpec(
        num_scalar_prefetch=1,
        grid=(NBLK,),
        in_specs=[
            pl.BlockSpec((RBLK, H), lambda b, be: (b, 0)),
            pl.BlockSpec((1, F, H), lambda b, be: (be[b], 0, 0)),
            pl.BlockSpec((1, F, H), lambda b, be: (be[b], 0, 0)),
            pl.BlockSpec((1, F, H), lambda b, be: (be[b], 0, 0)),
        ],
        out_specs=pl.BlockSpec((RBLK, H), lambda b, be: (b, 0)),
    )
    return pl.pallas_call(
        _gemm_body,
        grid_spec=grid_spec,
        out_shape=jax.ShapeDtypeStruct((NROWS, H), jnp.float32),
    )(bexp, xs, w1, v1, w2)


CCH = 16


def _combine_body(y_hbm, sortpos_hbm, wts_hbm, out_hbm,
                  i0, i1, w0b, w1b, buf0, buf1, ob, sem):
    cid = lax.axis_index("c")
    sid = lax.axis_index("s")
    wid = sid * NC + cid
    lane = _lane16()
    for q in range(T // CCH // NW):
        t0 = (wid + q * NW) * CCH
        pltpu.sync_copy(sortpos_hbm.at[pl.ds(t0, CCH)], i0)
        pltpu.sync_copy(sortpos_hbm.at[pl.ds(T + t0, CCH)], i1)
        pltpu.sync_copy(wts_hbm.at[0, pl.ds(t0, CCH)], w0b)
        pltpu.sync_copy(wts_hbm.at[1, pl.ds(t0, CCH)], w1b)
        cp0 = pltpu.async_copy(y_hbm.at[i0], buf0, sem)
        cp1 = pltpu.async_copy(y_hbm.at[i1], buf1, sem)
        cp0.wait()
        cp1.wait()
        w0v = w0b[...]
        w1v = w1b[...]
        for i in range(CCH):
            s0 = jnp.sum(jnp.where(lane == i, w0v, 0.0))
            s1 = jnp.sum(jnp.where(lane == i, w1v, 0.0))

            def body(c, _):
                sl = pl.ds(c * 16, 16)
                ob[i, sl] = buf0[i, sl] * s0 + buf1[i, sl] * s1
                return 0

            lax.fori_loop(0, H // 16, body, 0)
        pltpu.sync_copy(ob, out_hbm.at[pl.ds(t0, CCH)])


_combine = pl.kernel(
    _combine_body,
    out_type=jax.ShapeDtypeStruct((T, H), jnp.float32),
    mesh=_sc_mesh,
    compiler_params=pltpu.CompilerParams(needs_layout_passes=False),
    scratch_types=[
        pltpu.VMEM((CCH,), jnp.int32),
        pltpu.VMEM((CCH,), jnp.int32),
        pltpu.VMEM((CCH,), jnp.float32),
        pltpu.VMEM((CCH,), jnp.float32),
        pltpu.VMEM((CCH, H), jnp.float32),
        pltpu.VMEM((CCH, H), jnp.float32),
        pltpu.VMEM((CCH, H), jnp.float32),
        pltpu.SemaphoreType.DMA,
    ],
)


_SC_META = False
_SC_GATHER = False
_SC_COMBINE = False


def _meta_jnp(eids_flat):
    cnt = jnp.bincount(eids_flat, length=E)
    padded = ((cnt + (RBLK - 1)) // RBLK) * RBLK
    poff = jnp.cumsum(padded) - padded
    coff = jnp.cumsum(cnt) - cnt
    order = jnp.argsort(eids_flat, stable=True)
    e_of = eids_flat[order]
    ranks = jnp.arange(A, dtype=jnp.int32)
    pos_of_order = poff[e_of] + ranks - coff[e_of]
    sortpos = jnp.zeros((A,), jnp.int32).at[order].set(pos_of_order)
    sorted_tok = jnp.zeros((NROWS,), jnp.int32).at[pos_of_order].set(order & (T - 1))
    nr = jnp.sum(padded)
    blk = jnp.arange(NBLK, dtype=jnp.int32) * RBLK
    last_e = jnp.max(jnp.where(padded > 0, jnp.arange(E), 0)).astype(jnp.int32)
    inside = (blk[:, None] >= poff[None, :]) & (blk[:, None] < (poff + padded)[None, :])
    bexp = jnp.sum(jnp.where(inside, jnp.arange(E)[None, :], 0), axis=1).astype(jnp.int32)
    bexp = jnp.where(blk < nr, bexp, last_e)
    return sortpos, sorted_tok, bexp, None


@jax.jit
def kernel(hidden_states, router_w, w1, v1, w2):
    xf = hidden_states.reshape(T, H)
    rw_pad = jnp.zeros((LANES, H), jnp.float32).at[:E].set(router_w)

    eids, wts = _router(xf, rw_pad)
    eids_flat = eids[:TOP_K].reshape(A)
    if _SC_META:
        sortpos, sorted_tok, bexp, nrows = _meta(eids_flat)
        bexp = bexp[:NBLK]
    else:
        sortpos, sorted_tok, bexp, _ = _meta_jnp(eids_flat)
    if _SC_GATHER:
        xs = _gather(xf, sorted_tok)
    else:
        xs = xf[sorted_tok]
    y = _gemm(bexp, xs, w1, v1, w2)
    if _SC_COMBINE:
        out = _combine(y, sortpos, wts[:TOP_K])
    else:
        w2d = wts[:TOP_K]
        out = w2d[0][:, None] * y[sortpos[:T]] + w2d[1][:, None] * y[sortpos[T:]]
    return out.reshape(1, T, H)
